# baseline (speedup 1.0000x reference)
import jax
import jax.numpy as jnp
from jax import lax
from jax.experimental import pallas as pl
from jax.experimental.pallas import tpu as pltpu
from jax.experimental.pallas import tpu_sc as plsc

D = 4096
NC = 2
NS = 16
NW = NC * NS

CHUNK = 15


def _gather_kernel(B, b_per_w):
    n_full = b_per_w // CHUNK
    tail = b_per_w - n_full * CHUNK
    mesh = plsc.VectorSubcoreMesh(
        core_axis_name="c", subcore_axis_name="s", num_cores=NC, num_subcores=NS
    )

    n_chunks_pad = n_full + (1 if tail else 0)
    idx_len = n_chunks_pad * 16

    def body(pos_hbm, pe_hbm, out_hbm, idx_v, buf0, buf1, gs0, gs1, ss0, ss1):
        bufs = (buf0, buf1)
        gsems = (gs0, gs1)
        ssems = (ss0, ss1)

        wid = lax.axis_index("s") * NC + lax.axis_index("c")
        base = wid * b_per_w
        pltpu.sync_copy(pos_hbm.at[pl.ds(wid * idx_len, idx_len)], idx_v)

        def start_gather(g, slot, n=CHUNK):
            pltpu.async_copy(
                pe_hbm.at[idx_v.at[pl.ds(g * 16, n)]],
                bufs[slot].at[pl.ds(0, n)],
                gsems[slot],
            )

        def wait_gather(g, slot, n=CHUNK):
            pltpu.make_async_copy(
                pe_hbm.at[idx_v.at[pl.ds(g * 16, n)]],
                bufs[slot].at[pl.ds(0, n)],
                gsems[slot],
            ).wait()

        def start_store(g, slot, n=CHUNK):
            pltpu.async_copy(
                bufs[slot].at[pl.ds(0, n)],
                out_hbm.at[pl.ds(base + g * CHUNK, n)],
                ssems[slot],
            )

        def wait_store(slot, n=CHUNK):
            pltpu.make_async_copy(
                bufs[slot].at[pl.ds(0, n)],
                out_hbm.at[pl.ds(base, n)],
                ssems[slot],
            ).wait()

        start_gather(0, 0)

        def step(i, _):
            def run(slot):
                wait_gather(i, slot)
                start_store(i, slot)

                @pl.when(i + 1 < n_full)
                def _():
                    @pl.when(i >= 1)
                    def _():
                        wait_store(1 - slot)

                    start_gather(i + 1, 1 - slot)

            lax.cond(i % 2 == 0, lambda: run(0), lambda: run(1))
            return _

        lax.fori_loop(0, n_full, step, 0)

        tslot = n_full % 2
        if tail:
            wait_store(tslot)
            start_gather(n_full, tslot, tail)
            wait_gather(n_full, tslot, tail)
            start_store(n_full, tslot, tail)
        wait_store(1 - tslot)
        wait_store(tslot, tail if tail else CHUNK)

    return pl.kernel(
        body,
        out_type=jax.ShapeDtypeStruct((B, D), jnp.float32),
        mesh=mesh,
        compiler_params=pltpu.CompilerParams(use_tc_tiling_on_sc=False),
        scratch_types=(
            [pltpu.VMEM((idx_len,), jnp.int32)]
            + [pltpu.VMEM((CHUNK, D), jnp.float32) for _ in range(2)]
            + [pltpu.SemaphoreType.DMA for _ in range(4)]
        ),
    )


def _arrange_idx(flat_pos, b_per_w):
    n_full = b_per_w // CHUNK
    tail = b_per_w - n_full * CHUNK
    n_chunks = n_full + (1 if tail else 0)
    w = flat_pos.reshape(NW, b_per_w)
    w = jnp.pad(w, ((0, 0), (0, n_chunks * CHUNK - b_per_w)))
    w = w.reshape(NW, n_chunks, CHUNK)
    w = jnp.pad(w, ((0, 0), (0, 0), (0, 16 - CHUNK)))
    return w.reshape(NW * n_chunks * 16)


def kernel(pos, pe):
    batch, seq = pos.shape
    B = batch * seq
    flat_pos = pos.reshape(B).astype(jnp.int32)
    arranged = _arrange_idx(flat_pos, B // NW)
    out = _gather_kernel(B, B // NW)(arranged, pe)
    return out.reshape(batch, seq, D)


# speedup vs baseline: 2.6367x; 2.6367x over previous
import jax
import jax.numpy as jnp
from jax import lax
from jax.experimental import pallas as pl
from jax.experimental.pallas import tpu as pltpu
from jax.experimental.pallas import tpu_sc as plsc

D = 4096
NC = 2
NS = 16
NW = NC * NS

CHUNK = 8
NBUF = 3


def _gather_kernel(B, b_per_w):
    n_chunks = b_per_w // CHUNK
    mesh = plsc.VectorSubcoreMesh(
        core_axis_name="c", subcore_axis_name="s", num_cores=NC, num_subcores=NS
    )

    def body(pos_hbm, pe_hbm, out_hbm, idx_v, *bufs_and_sems):
        bufs = bufs_and_sems[:NBUF]
        gsems = bufs_and_sems[NBUF:2 * NBUF]
        ssems = bufs_and_sems[2 * NBUF:3 * NBUF]

        wid = lax.axis_index("s") * NC + lax.axis_index("c")
        base = wid * b_per_w
        pltpu.sync_copy(pos_hbm.at[pl.ds(base, b_per_w)], idx_v)

        def start_gather(g, slot):
            pltpu.async_copy(
                pe_hbm.at[idx_v.at[pl.ds(g * CHUNK, CHUNK)]], bufs[slot], gsems[slot]
            )

        def wait_gather(g, slot):
            pltpu.make_async_copy(
                pe_hbm.at[idx_v.at[pl.ds(g * CHUNK, CHUNK)]], bufs[slot], gsems[slot]
            ).wait()

        def start_store(g, slot):
            pltpu.async_copy(
                bufs[slot], out_hbm.at[pl.ds(base + g * CHUNK, CHUNK)], ssems[slot]
            )

        def wait_store(slot):
            pltpu.make_async_copy(
                bufs[slot], out_hbm.at[pl.ds(base, CHUNK)], ssems[slot]
            ).wait()

        start_gather(0, 0)
        start_gather(1, 1)

        def step(h, _):
            def run(slot):
                nslot = (slot + 2) % NBUF
                wait_gather(h, slot)

                @pl.when(h + 2 < n_chunks)
                def _():
                    @pl.when(h >= 1)
                    def _():
                        wait_store(nslot)

                    start_gather(h + 2, nslot)

                start_store(h, slot)

            lax.switch(h % NBUF, [lambda: run(0), lambda: run(1), lambda: run(2)])
            return _

        lax.fori_loop(0, n_chunks, step, 0)

        for slot in range(NBUF):
            wait_store(slot)

    return pl.kernel(
        body,
        out_type=jax.ShapeDtypeStruct((B, D), jnp.float32),
        mesh=mesh,
        scratch_types=(
            [pltpu.VMEM((b_per_w,), jnp.int32)]
            + [pltpu.VMEM((CHUNK, D), jnp.float32) for _ in range(NBUF)]
            + [pltpu.SemaphoreType.DMA for _ in range(2 * NBUF)]
        ),
    )


def kernel(pos, pe):
    batch, seq = pos.shape
    B = batch * seq
    flat_pos = pos.reshape(B).astype(jnp.int32)
    out = _gather_kernel(B, B // NW)(flat_pos, pe)
    return out.reshape(batch, seq, D)
